# baseline (speedup 1.0000x reference)
import functools

import jax
import jax.numpy as jnp
from jax import lax
from jax.experimental import pallas as pl
from jax.experimental.pallas import tpu as pltpu
from jax.experimental.pallas import tpu_sc as plsc

_info = plsc.get_sparse_core_info()
_NC, _NS = _info.num_cores, _info.num_subcores
_NW = _NC * _NS


def _make_gather(B1, S, D, V):
    bpt = B1 // _NW
    mesh = plsc.VectorSubcoreMesh(core_axis_name="c", subcore_axis_name="s")

    @functools.partial(
        pl.kernel,
        mesh=mesh,
        out_type=jax.ShapeDtypeStruct((S, D, B1), jnp.float32),
        compiler_params=pltpu.CompilerParams(
            use_tc_tiling_on_sc=False, needs_layout_passes=False
        ),
        scratch_types=[
            pltpu.VMEM((bpt, S), jnp.int32),
            pltpu.VMEM((2, bpt), jnp.int32),
            pltpu.VMEM((2, bpt, D), jnp.float32),
            pltpu.VMEM((D, bpt), jnp.float32),
            pltpu.SemaphoreType.DMA((2,)),
            pltpu.SemaphoreType.DMA,
        ],
    )
    def k(idx_hbm, table_hbm, out_hbm, idxblk, idxcol, rows, rowst, gsem, wsem):
        wid = lax.axis_index("s") * _NC + lax.axis_index("c")
        b0 = wid * bpt
        pltpu.sync_copy(idx_hbm.at[pl.ds(b0, bpt)], idxblk)

        def build_and_start(s, buf):
            scol = jnp.full((16,), s, jnp.int32)
            for i in range(bpt // 16):
                bvec = lax.iota(jnp.int32, 16) + (i * 16)
                idxcol[buf, pl.ds(i * 16, 16)] = plsc.load_gather(
                    idxblk, [bvec, scol]
                )
            pltpu.make_async_copy(
                table_hbm.at[idxcol.at[buf]], rows.at[buf], gsem.at[buf]
            ).start()

        build_and_start(0, 0)
        build_and_start(1, 1)

        def body(s, _):
            buf = lax.rem(s, 2)
            pltpu.make_async_copy(
                table_hbm.at[idxcol.at[buf]], rows.at[buf], gsem.at[buf]
            ).wait()

            @pl.when(s >= 1)
            def _():
                pltpu.make_async_copy(
                    rowst, out_hbm.at[0, :, pl.ds(b0, bpt)], wsem
                ).wait()

            def tp(d, _):
                for i in range(bpt // 16):
                    bvec = lax.iota(jnp.int32, 16) + (i * 16)
                    dvec = jnp.full((16,), d, jnp.int32)
                    rowst[d, pl.ds(i * 16, 16)] = plsc.load_gather(
                        rows.at[buf], [bvec, dvec]
                    )
                return 0

            lax.fori_loop(0, D, tp, 0)

            pltpu.make_async_copy(
                rowst, out_hbm.at[s, :, pl.ds(b0, bpt)], wsem
            ).start()

            @pl.when(s + 2 < S)
            def _():
                build_and_start(s + 2, buf)

            return 0

        lax.fori_loop(0, S, body, 0)

        pltpu.make_async_copy(
            rowst, out_hbm.at[0, :, pl.ds(b0, bpt)], wsem
        ).wait()

    return k


def kernel(road_ids, table):
    B1, S = road_ids.shape
    V, D = table.shape
    outt = _make_gather(B1, S, D, V)(road_ids.astype(jnp.int32), table)
    return outt.transpose(2, 0, 1)


# speedup vs baseline: 1.6819x; 1.6819x over previous
import functools

import jax
import jax.numpy as jnp
from jax import lax
from jax.experimental import pallas as pl
from jax.experimental.pallas import tpu as pltpu
from jax.experimental.pallas import tpu_sc as plsc

NUM_FEATURE = 64

_info = plsc.get_sparse_core_info()
_NC, _NS = _info.num_cores, _info.num_subcores
_NW = _NC * _NS

_NBUF = 3


def _make_gather(B, V, D, chunk):
    b_per_w = B // _NW
    n_chunks = b_per_w // chunk
    mesh = plsc.VectorSubcoreMesh(core_axis_name="c", subcore_axis_name="s")

    @functools.partial(
        pl.kernel,
        mesh=mesh,
        out_type=jax.ShapeDtypeStruct((B, D), jnp.float32),
        compiler_params=pltpu.CompilerParams(use_tc_tiling_on_sc=False),
        scratch_types=[
            pltpu.VMEM((b_per_w,), jnp.int32),
            pltpu.VMEM((_NBUF, chunk, D), jnp.float32),
            pltpu.SemaphoreType.DMA((_NBUF,)),
            pltpu.SemaphoreType.DMA((_NBUF,)),
        ],
    )
    def k(idx_hbm, table_hbm, out_hbm, idx_all, rows, gsem, wsem):
        wid = lax.axis_index("s") * _NC + lax.axis_index("c")
        base = wid * b_per_w
        pltpu.sync_copy(idx_hbm.at[pl.ds(base, b_per_w)], idx_all)

        def gather_start(g, b):
            pltpu.make_async_copy(
                table_hbm.at[idx_all.at[pl.ds(g * chunk, chunk)]],
                rows.at[b],
                gsem.at[b],
            ).start()

        gather_start(0, 0)
        gather_start(1, 1)

        def body(g, _):
            b = lax.rem(g, _NBUF)
            b2 = lax.rem(g + 2, _NBUF)
            @pl.when(g >= 1)
            def _():
                pltpu.make_async_copy(
                    rows.at[b2],
                    out_hbm.at[pl.ds(base, chunk)],
                    wsem.at[b2],
                ).wait()

            @pl.when(g + 2 < n_chunks)
            def _():
                gather_start(g + 2, b2)

            pltpu.make_async_copy(
                table_hbm.at[idx_all.at[pl.ds(0, chunk)]],
                rows.at[b],
                gsem.at[b],
            ).wait()
            pltpu.make_async_copy(
                rows.at[b],
                out_hbm.at[pl.ds(base + g * chunk, chunk)],
                wsem.at[b],
            ).start()
            return 0

        lax.fori_loop(0, n_chunks, body, 0)

        bl = (n_chunks - 1) % _NBUF
        pltpu.make_async_copy(
            rows.at[bl],
            out_hbm.at[pl.ds(base, chunk)],
            wsem.at[bl],
        ).wait()

    return k


def kernel(road_ids, table):
    orig_shape = road_ids.shape
    idx = road_ids.reshape(-1).astype(jnp.int32)
    B = idx.shape[0]
    V, D = table.shape
    out = _make_gather(B, V, D, 512)(idx, table)
    return out.reshape(*orig_shape, D)
